# baseline (speedup 1.0000x reference)
import functools

import jax
import jax.numpy as jnp
from jax import lax
from jax.experimental import pallas as pl
from jax.experimental.pallas import tpu as pltpu
from jax.experimental.pallas import tpu_sc as plsc

N = 10000
E = 160000
D = 256
H = 128
C = 32
B = 16

NP = 10240
PAD_NODE = N
NW = 32
CHUNK = 128
EPW = 5120
NCH = EPW // CHUNK
EP = NW * EPW
ECH = E // CHUNK
RCH31 = ECH - 31 * NCH
RPT = NP // 16
GCH = 64
NGC = EPW // GCH
NBUF = 4

_mesh = plsc.VectorSubcoreMesh(core_axis_name="c", subcore_axis_name="s")


@functools.partial(
    pl.kernel,
    mesh=_mesh,
    out_type=jax.ShapeDtypeStruct((2, NP), jnp.float32),
    scratch_types=[
        pltpu.VMEM((NCH, CHUNK), jnp.int32),
        pltpu.VMEM((CHUNK,), jnp.float32),
        pltpu.VMEM((RPT,), jnp.float32),
        pltpu.VMEM_SHARED((NP,), jnp.float32),
        pltpu.SemaphoreType.DMA,
    ],
)
def _sc_degree(dst_hbm, out_hbm, idx_v, ones_v, buf_v, shared_deg, sem):
    cid = lax.axis_index("c")
    sid = lax.axis_index("s")
    wid = sid * 2 + cid

    one16 = jnp.ones((16,), jnp.float32)
    for k in range(CHUNK // 16):
        ones_v[pl.ds(k * 16, 16)] = one16

    z16 = jnp.zeros((16,), jnp.float32)

    def _zero(i, carry):
        buf_v[pl.ds(i * 16, 16)] = z16
        return carry

    lax.fori_loop(0, RPT // 16, _zero, 0)
    pltpu.sync_copy(buf_v, shared_deg.at[pl.ds(sid * RPT, RPT)])
    plsc.subcore_barrier()

    pltpu.sync_copy(dst_hbm.at[wid], idx_v)

    def _body(j, carry):
        pltpu.sync_copy(ones_v, shared_deg.at[idx_v.at[j]], add=True)
        return carry

    lax.fori_loop(0, NCH, _body, 0)
    plsc.subcore_barrier()

    pltpu.sync_copy(shared_deg.at[pl.ds(sid * RPT, RPT)], buf_v)
    pltpu.sync_copy(buf_v, out_hbm.at[cid, pl.ds(sid * RPT, RPT)])


@functools.partial(
    pl.kernel,
    mesh=_mesh,
    out_type=jax.ShapeDtypeStruct((2, NP, H), jnp.float32),
    scratch_types=[
        pltpu.VMEM((NCH, CHUNK), jnp.int32),
        pltpu.VMEM((NCH, CHUNK), jnp.int32),
        pltpu.VMEM((GCH, H), jnp.float32),
        pltpu.VMEM((GCH, H), jnp.float32),
        pltpu.VMEM((GCH, H), jnp.float32),
        pltpu.VMEM((GCH, H), jnp.float32),
        pltpu.VMEM_SHARED((NP, H), jnp.float32),
        pltpu.SemaphoreType.DMA,
        pltpu.SemaphoreType.DMA,
        pltpu.SemaphoreType.DMA,
        pltpu.SemaphoreType.DMA,
        pltpu.SemaphoreType.DMA,
        pltpu.SemaphoreType.DMA,
        pltpu.SemaphoreType.DMA,
        pltpu.SemaphoreType.DMA,
    ],
)
def _sc_aggregate(hs_hbm, src_hbm, dst_hbm, out_hbm,
                  src_v, dst_v, rows0, rows1, rows2, rows3, shared_acc,
                  g0, g1, g2, g3, s0, s1, s2, s3):
    cid = lax.axis_index("c")
    sid = lax.axis_index("s")
    wid = sid * 2 + cid
    rows = (rows0, rows1, rows2, rows3)
    semg = (g0, g1, g2, g3)
    sems = (s0, s1, s2, s3)

    z16 = jnp.zeros((16,), jnp.float32)

    def _zrow(i, carry):
        for k in range(H // 16):
            rows0[i, pl.ds(k * 16, 16)] = z16
        return carry

    lax.fori_loop(0, GCH, _zrow, 0)
    for k in range(RPT // GCH):
        pltpu.sync_copy(rows0, shared_acc.at[pl.ds(sid * RPT + k * GCH, GCH)])
    pltpu.sync_copy(src_hbm.at[wid], src_v)
    pltpu.sync_copy(dst_hbm.at[wid], dst_v)
    plsc.subcore_barrier()

    def _sidx(idx_ref, j):
        return idx_ref.at[j // 2, pl.ds((j % 2) * GCH, GCH)]

    for b in range(NBUF - 1):
        pltpu.async_copy(hs_hbm.at[_sidx(src_v, b)], rows[b], semg[b])

    def _body(i, carry):
        for b in range(NBUF):
            j = NBUF * i + b
            o = (b + NBUF - 1) % NBUF

            @pl.when(j >= 1)
            def _wait_prev_scatter():
                pltpu.make_async_copy(rows[o], shared_acc.at[_sidx(dst_v, j)],
                                      sems[o]).wait()

            @pl.when(j + NBUF - 1 < NGC)
            def _issue_gather():
                pltpu.async_copy(hs_hbm.at[_sidx(src_v, j + NBUF - 1)],
                                 rows[o], semg[o])

            pltpu.make_async_copy(hs_hbm.at[_sidx(src_v, j)], rows[b],
                                  semg[b]).wait()
            pltpu.async_copy(rows[b], shared_acc.at[_sidx(dst_v, j)], sems[b],
                             add=True)
        return carry

    lax.fori_loop(0, NGC // NBUF, _body, 0)
    pltpu.make_async_copy(rows[(NGC - 1) % NBUF],
                          shared_acc.at[_sidx(dst_v, NGC - 1)],
                          sems[(NGC - 1) % NBUF]).wait()
    plsc.subcore_barrier()

    for k in range(RPT // CHUNK):
        r = sid * RPT + k * CHUNK
        pltpu.sync_copy(shared_acc.at[pl.ds(r, CHUNK)],
                        out_hbm.at[cid, pl.ds(r, CHUNK)])


_RB = 2560
_NBLK = NP // _RB


def _dis_block(deg_ref):
    return lax.rsqrt(deg_ref[0] + deg_ref[1] + 1.0)


def _tc1a_body(x_ref, w_ref, h_ref):
    h_ref[...] = jnp.dot(x_ref[...], w_ref[...],
                         preferred_element_type=jnp.float32)


_tc1a = pl.pallas_call(
    _tc1a_body,
    grid=(_NBLK,),
    in_specs=[
        pl.BlockSpec((_RB, D), lambda i: (i, 0)),
        pl.BlockSpec((D, H), lambda i: (0, 0)),
    ],
    out_specs=pl.BlockSpec((_RB, H), lambda i: (i, 0)),
    out_shape=jax.ShapeDtypeStruct((NP, H), jnp.float32),
)


def _tc1b_body(h_ref, deg_ref, hs_ref):
    dis = _dis_block(deg_ref)
    hs_ref[...] = h_ref[...] * jnp.broadcast_to(dis, (_RB, H))


_tc1b = pl.pallas_call(
    _tc1b_body,
    grid=(_NBLK,),
    in_specs=[
        pl.BlockSpec((_RB, H), lambda i: (i, 0)),
        pl.BlockSpec((2, _RB, 1), lambda i: (0, i, 0)),
    ],
    out_specs=pl.BlockSpec((_RB, H), lambda i: (i, 0)),
    out_shape=jax.ShapeDtypeStruct((NP, H), jnp.float32),
)


def _tc_mid_body(agg_ref, hsp_ref, deg_ref, b_ref, w_ref, out_ref):
    disb = jnp.broadcast_to(_dis_block(deg_ref), (_RB, H))
    t = (agg_ref[0] + agg_ref[1] + hsp_ref[...]) * disb + b_ref[...]
    t = jnp.where(t >= 0, t, 0.01 * t)
    out_ref[...] = jnp.dot(t, w_ref[...], preferred_element_type=jnp.float32) * disb


_tc_mid = pl.pallas_call(
    _tc_mid_body,
    grid=(_NBLK,),
    in_specs=[
        pl.BlockSpec((2, _RB, H), lambda i: (0, i, 0)),
        pl.BlockSpec((_RB, H), lambda i: (i, 0)),
        pl.BlockSpec((2, _RB, 1), lambda i: (0, i, 0)),
        pl.BlockSpec((1, H), lambda i: (0, 0)),
        pl.BlockSpec((H, H), lambda i: (0, 0)),
    ],
    out_specs=pl.BlockSpec((_RB, H), lambda i: (i, 0)),
    out_shape=jax.ShapeDtypeStruct((NP, H), jnp.float32),
)


def _tc_final_body(agg_ref, hsp_ref, deg_ref, b_ref, batch_ref,
                   wfc_ref, bfc_ref, out_ref):
    dis = lax.rsqrt(deg_ref[0] + deg_ref[1] + 1.0)
    h3 = ((agg_ref[0] + agg_ref[1] + hsp_ref[...])
          * jnp.broadcast_to(dis, (NP, H)) + b_ref[...])
    bt = batch_ref[...]
    iot = lax.broadcasted_iota(jnp.int32, (B, NP), 0)
    onehot = jnp.where(bt == iot, 1.0, 0.0).astype(jnp.float32)
    sums = jnp.dot(onehot, h3, preferred_element_type=jnp.float32)
    cnt = jnp.sum(onehot, axis=1, keepdims=True)
    pooled = sums / jnp.maximum(cnt, 1.0)
    embed = jnp.dot(pooled, wfc_ref[...],
                    preferred_element_type=jnp.float32) + bfc_ref[...]
    m = jnp.max(embed, axis=1, keepdims=True)
    ex = jnp.exp(embed - m)
    lse = jnp.log(jnp.sum(ex, axis=1, keepdims=True)) + m
    out_ref[...] = embed - lse


_tc_final = pl.pallas_call(
    _tc_final_body,
    out_shape=jax.ShapeDtypeStruct((B, C), jnp.float32),
)


def kernel(x, edge_index, batch, W1, b1, W2, b2, W3, b3, Wfc, bfc):
    pad = jnp.arange(EP - E, dtype=jnp.int32)
    src_p = jnp.concatenate([edge_index[0].astype(jnp.int32), pad % N])
    dst_p = jnp.concatenate([edge_index[1].astype(jnp.int32),
                             pad % (NP - N) + N])
    src_p = src_p.reshape(NW, NCH, CHUNK)
    dst_p = dst_p.reshape(NW, NCH, CHUNK)
    x_p = jnp.pad(x, ((0, NP - N), (0, 0)))
    batch_p = jnp.pad(batch, (0, NP - N), constant_values=-1).reshape(1, NP)

    deg2 = _sc_degree(dst_p).reshape(2, NP, 1)
    hs1 = _tc1b(_tc1a(x_p, W1), deg2)
    agg1 = _sc_aggregate(hs1, src_p, dst_p)
    hs2 = _tc_mid(agg1, hs1, deg2, b1.reshape(1, H), W2)
    agg2 = _sc_aggregate(hs2, src_p, dst_p)
    hs3 = _tc_mid(agg2, hs2, deg2, b2.reshape(1, H), W3)
    agg3 = _sc_aggregate(hs3, src_p, dst_p)
    logits = _tc_final(agg3, hs3, deg2, b3.reshape(1, H), batch_p,
                       Wfc, bfc.reshape(1, C))
    return logits


# speedup vs baseline: 1.0181x; 1.0181x over previous
import functools

import jax
import jax.numpy as jnp
from jax import lax
from jax.experimental import pallas as pl
from jax.experimental.pallas import tpu as pltpu
from jax.experimental.pallas import tpu_sc as plsc

N = 10000
E = 160000
D = 256
H = 128
C = 32
B = 16

NP = 10240
PAD_NODE = N
NW = 32
CHUNK = 128
EPW = 5120
NCH = EPW // CHUNK
EP = NW * EPW
ECH = E // CHUNK
RCH31 = ECH - 31 * NCH
RPT = NP // 16
GCH = 64
NGC = EPW // GCH
NBUF = 4

_mesh = plsc.VectorSubcoreMesh(core_axis_name="c", subcore_axis_name="s")


@functools.partial(
    pl.kernel,
    mesh=_mesh,
    out_type=jax.ShapeDtypeStruct((2, NP), jnp.float32),
    scratch_types=[
        pltpu.VMEM((NCH, CHUNK), jnp.int32),
        pltpu.VMEM((CHUNK,), jnp.float32),
        pltpu.VMEM((RPT,), jnp.float32),
        pltpu.VMEM_SHARED((NP,), jnp.float32),
        pltpu.SemaphoreType.DMA,
    ],
)
def _sc_degree(dst_hbm, out_hbm, idx_v, ones_v, buf_v, shared_deg, sem):
    cid = lax.axis_index("c")
    sid = lax.axis_index("s")
    wid = sid * 2 + cid

    one16 = jnp.ones((16,), jnp.float32)
    for k in range(CHUNK // 16):
        ones_v[pl.ds(k * 16, 16)] = one16

    z16 = jnp.zeros((16,), jnp.float32)

    def _zero(i, carry):
        buf_v[pl.ds(i * 16, 16)] = z16
        return carry

    lax.fori_loop(0, RPT // 16, _zero, 0)
    pltpu.sync_copy(buf_v, shared_deg.at[pl.ds(sid * RPT, RPT)])
    plsc.subcore_barrier()

    pltpu.sync_copy(dst_hbm.at[wid], idx_v)

    def _body(j, carry):
        pltpu.sync_copy(ones_v, shared_deg.at[idx_v.at[j]], add=True)
        return carry

    lax.fori_loop(0, NCH, _body, 0)
    plsc.subcore_barrier()

    pltpu.sync_copy(shared_deg.at[pl.ds(sid * RPT, RPT)], buf_v)
    pltpu.sync_copy(buf_v, out_hbm.at[cid, pl.ds(sid * RPT, RPT)])


@functools.partial(
    pl.kernel,
    mesh=_mesh,
    out_type=jax.ShapeDtypeStruct((2, NP, H), jnp.float32),
    scratch_types=[
        pltpu.VMEM((NCH, CHUNK), jnp.int32),
        pltpu.VMEM((NCH, CHUNK), jnp.int32),
        pltpu.VMEM((GCH, H), jnp.float32),
        pltpu.VMEM((GCH, H), jnp.float32),
        pltpu.VMEM((GCH, H), jnp.float32),
        pltpu.VMEM((GCH, H), jnp.float32),
        pltpu.VMEM_SHARED((NP, H), jnp.float32),
        pltpu.SemaphoreType.DMA,
        pltpu.SemaphoreType.DMA,
        pltpu.SemaphoreType.DMA,
        pltpu.SemaphoreType.DMA,
        pltpu.SemaphoreType.DMA,
        pltpu.SemaphoreType.DMA,
        pltpu.SemaphoreType.DMA,
        pltpu.SemaphoreType.DMA,
    ],
)
def _sc_aggregate(hs_hbm, src_hbm, dst_hbm, out_hbm,
                  src_v, dst_v, rows0, rows1, rows2, rows3, shared_acc,
                  g0, g1, g2, g3, s0, s1, s2, s3):
    cid = lax.axis_index("c")
    sid = lax.axis_index("s")
    wid = sid * 2 + cid
    rows = (rows0, rows1, rows2, rows3)
    semg = (g0, g1, g2, g3)
    sems = (s0, s1, s2, s3)

    z16 = jnp.zeros((16,), jnp.float32)

    def _zrow(i, carry):
        for k in range(H // 16):
            rows0[i, pl.ds(k * 16, 16)] = z16
        return carry

    lax.fori_loop(0, GCH, _zrow, 0)
    for k in range(RPT // GCH):
        pltpu.sync_copy(rows0, shared_acc.at[pl.ds(sid * RPT + k * GCH, GCH)])
    pltpu.sync_copy(src_hbm.at[wid], src_v)
    pltpu.sync_copy(dst_hbm.at[wid], dst_v)
    plsc.subcore_barrier()

    def _sidx(idx_ref, j):
        return idx_ref.at[j // 2, pl.ds((j % 2) * GCH, GCH)]

    for b in range(NBUF - 1):
        pltpu.async_copy(hs_hbm.at[_sidx(src_v, b)], rows[b], semg[b])

    def _body(i, carry):
        for b in range(NBUF):
            j = NBUF * i + b
            o = (b + NBUF - 1) % NBUF

            @pl.when(j >= 1)
            def _wait_prev_scatter():
                pltpu.make_async_copy(rows[o], shared_acc.at[_sidx(dst_v, j)],
                                      sems[o]).wait()

            @pl.when(j + NBUF - 1 < NGC)
            def _issue_gather():
                pltpu.async_copy(hs_hbm.at[_sidx(src_v, j + NBUF - 1)],
                                 rows[o], semg[o])

            pltpu.make_async_copy(hs_hbm.at[_sidx(src_v, j)], rows[b],
                                  semg[b]).wait()
            pltpu.async_copy(rows[b], shared_acc.at[_sidx(dst_v, j)], sems[b],
                             add=True)
        return carry

    lax.fori_loop(0, NGC // NBUF, _body, 0)
    pltpu.make_async_copy(rows[(NGC - 1) % NBUF],
                          shared_acc.at[_sidx(dst_v, NGC - 1)],
                          sems[(NGC - 1) % NBUF]).wait()
    plsc.subcore_barrier()

    for k in range(RPT // CHUNK):
        r = sid * RPT + k * CHUNK
        pltpu.sync_copy(shared_acc.at[pl.ds(r, CHUNK)],
                        out_hbm.at[cid, pl.ds(r, CHUNK)])


_RB = 2560
_NBLK = NP // _RB


def _dis_block(deg_ref):
    return lax.rsqrt(deg_ref[0] + deg_ref[1] + 1.0)


def _tc1_body(x_ref, w_ref, deg_ref, hs_ref):
    dis = _dis_block(deg_ref)
    h = jnp.dot(x_ref[...], w_ref[...], preferred_element_type=jnp.float32)
    hs_ref[...] = h * jnp.broadcast_to(dis, (_RB, H))


_tc1 = pl.pallas_call(
    _tc1_body,
    grid=(_NBLK,),
    in_specs=[
        pl.BlockSpec((_RB, D), lambda i: (i, 0)),
        pl.BlockSpec((D, H), lambda i: (0, 0)),
        pl.BlockSpec((2, _RB, 1), lambda i: (0, i, 0)),
    ],
    out_specs=pl.BlockSpec((_RB, H), lambda i: (i, 0)),
    out_shape=jax.ShapeDtypeStruct((NP, H), jnp.float32),
)


def _tc_mid_body(agg_ref, hsp_ref, deg_ref, b_ref, w_ref, out_ref):
    disb = jnp.broadcast_to(_dis_block(deg_ref), (_RB, H))
    t = (agg_ref[0] + agg_ref[1] + hsp_ref[...]) * disb + b_ref[...]
    t = jnp.where(t >= 0, t, 0.01 * t)
    out_ref[...] = jnp.dot(t, w_ref[...], preferred_element_type=jnp.float32) * disb


_tc_mid = pl.pallas_call(
    _tc_mid_body,
    grid=(_NBLK,),
    in_specs=[
        pl.BlockSpec((2, _RB, H), lambda i: (0, i, 0)),
        pl.BlockSpec((_RB, H), lambda i: (i, 0)),
        pl.BlockSpec((2, _RB, 1), lambda i: (0, i, 0)),
        pl.BlockSpec((1, H), lambda i: (0, 0)),
        pl.BlockSpec((H, H), lambda i: (0, 0)),
    ],
    out_specs=pl.BlockSpec((_RB, H), lambda i: (i, 0)),
    out_shape=jax.ShapeDtypeStruct((NP, H), jnp.float32),
)


def _tc_final_body(agg_ref, hsp_ref, deg_ref, b_ref, batch_ref,
                   wfc_ref, bfc_ref, out_ref):
    dis = lax.rsqrt(deg_ref[0] + deg_ref[1] + 1.0)
    h3 = ((agg_ref[0] + agg_ref[1] + hsp_ref[...])
          * jnp.broadcast_to(dis, (NP, H)) + b_ref[...])
    bt = batch_ref[...]
    iot = lax.broadcasted_iota(jnp.int32, (B, NP), 0)
    onehot = jnp.where(bt == iot, 1.0, 0.0).astype(jnp.float32)
    sums = jnp.dot(onehot, h3, preferred_element_type=jnp.float32)
    cnt = jnp.sum(onehot, axis=1, keepdims=True)
    pooled = sums / jnp.maximum(cnt, 1.0)
    embed = jnp.dot(pooled, wfc_ref[...],
                    preferred_element_type=jnp.float32) + bfc_ref[...]
    m = jnp.max(embed, axis=1, keepdims=True)
    ex = jnp.exp(embed - m)
    lse = jnp.log(jnp.sum(ex, axis=1, keepdims=True)) + m
    out_ref[...] = embed - lse


_tc_final = pl.pallas_call(
    _tc_final_body,
    out_shape=jax.ShapeDtypeStruct((B, C), jnp.float32),
)


def kernel(x, edge_index, batch, W1, b1, W2, b2, W3, b3, Wfc, bfc):
    pad = jnp.arange(EP - E, dtype=jnp.int32)
    src_p = jnp.concatenate([edge_index[0].astype(jnp.int32), pad % N])
    dst_p = jnp.concatenate([edge_index[1].astype(jnp.int32),
                             pad % (NP - N) + N])
    src_p = src_p.reshape(NW, NCH, CHUNK)
    dst_p = dst_p.reshape(NW, NCH, CHUNK)
    x_p = jnp.pad(x, ((0, NP - N), (0, 0)))
    batch_p = jnp.pad(batch, (0, NP - N), constant_values=-1).reshape(1, NP)

    deg2 = _sc_degree(dst_p).reshape(2, NP, 1)
    hs1 = _tc1(x_p, W1, deg2)
    agg1 = _sc_aggregate(hs1, src_p, dst_p)
    hs2 = _tc_mid(agg1, hs1, deg2, b1.reshape(1, H), W2)
    agg2 = _sc_aggregate(hs2, src_p, dst_p)
    hs3 = _tc_mid(agg2, hs2, deg2, b2.reshape(1, H), W3)
    agg3 = _sc_aggregate(hs3, src_p, dst_p)
    logits = _tc_final(agg3, hs3, deg2, b3.reshape(1, H), batch_p,
                       Wfc, bfc.reshape(1, C))
    return logits


# speedup vs baseline: 1.0194x; 1.0013x over previous
import functools

import jax
import jax.numpy as jnp
from jax import lax
from jax.experimental import pallas as pl
from jax.experimental.pallas import tpu as pltpu
from jax.experimental.pallas import tpu_sc as plsc

N = 10000
E = 160000
D = 256
H = 128
C = 32
B = 16

NP = 10240
NW = 32
CHUNK = 128
EPW = 5120
NCH = EPW // CHUNK
EP = NW * EPW
RPT = NP // 16
GCH = 64
NGC = EPW // GCH
NBUF = 4

_mesh = plsc.VectorSubcoreMesh(core_axis_name="c", subcore_axis_name="s")


@functools.partial(
    pl.kernel,
    mesh=_mesh,
    out_type=jax.ShapeDtypeStruct((2, NP), jnp.float32),
    scratch_types=[
        pltpu.VMEM((NCH, CHUNK), jnp.int32),
        pltpu.VMEM((CHUNK,), jnp.float32),
        pltpu.VMEM((RPT,), jnp.float32),
        pltpu.VMEM_SHARED((NP,), jnp.float32),
        pltpu.SemaphoreType.DMA,
    ],
)
def _sc_degree(dst_hbm, out_hbm, idx_v, ones_v, buf_v, shared_deg, sem):
    cid = lax.axis_index("c")
    sid = lax.axis_index("s")
    wid = sid * 2 + cid

    one16 = jnp.ones((16,), jnp.float32)
    for k in range(CHUNK // 16):
        ones_v[pl.ds(k * 16, 16)] = one16

    z16 = jnp.zeros((16,), jnp.float32)

    def _zero(i, carry):
        buf_v[pl.ds(i * 16, 16)] = z16
        return carry

    lax.fori_loop(0, RPT // 16, _zero, 0)
    pltpu.sync_copy(buf_v, shared_deg.at[pl.ds(sid * RPT, RPT)])
    plsc.subcore_barrier()

    pltpu.sync_copy(dst_hbm.at[wid], idx_v)

    def _body(j, carry):
        pltpu.sync_copy(ones_v, shared_deg.at[idx_v.at[j]], add=True)
        return carry

    lax.fori_loop(0, NCH, _body, 0)
    plsc.subcore_barrier()

    pltpu.sync_copy(shared_deg.at[pl.ds(sid * RPT, RPT)], buf_v)
    pltpu.sync_copy(buf_v, out_hbm.at[cid, pl.ds(sid * RPT, RPT)])


@functools.partial(
    pl.kernel,
    mesh=_mesh,
    out_type=jax.ShapeDtypeStruct((2, NP, H), jnp.float32),
    scratch_types=[
        pltpu.VMEM((NCH, CHUNK), jnp.int32),
        pltpu.VMEM((NCH, CHUNK), jnp.int32),
        pltpu.VMEM((GCH, H), jnp.float32),
        pltpu.VMEM((GCH, H), jnp.float32),
        pltpu.VMEM((GCH, H), jnp.float32),
        pltpu.VMEM((GCH, H), jnp.float32),
        pltpu.VMEM_SHARED((NP, H), jnp.float32),
        pltpu.SemaphoreType.DMA,
        pltpu.SemaphoreType.DMA,
        pltpu.SemaphoreType.DMA,
        pltpu.SemaphoreType.DMA,
        pltpu.SemaphoreType.DMA,
        pltpu.SemaphoreType.DMA,
        pltpu.SemaphoreType.DMA,
        pltpu.SemaphoreType.DMA,
    ],
)
def _sc_aggregate(hs_hbm, src_hbm, dst_hbm, out_hbm,
                  src_v, dst_v, rows0, rows1, rows2, rows3, shared_acc,
                  g0, g1, g2, g3, s0, s1, s2, s3):
    cid = lax.axis_index("c")
    sid = lax.axis_index("s")
    wid = sid * 2 + cid
    rows = (rows0, rows1, rows2, rows3)
    semg = (g0, g1, g2, g3)
    sems = (s0, s1, s2, s3)

    z16 = jnp.zeros((16,), jnp.float32)

    def _zrow(i, carry):
        for k in range(H // 16):
            rows0[i, pl.ds(k * 16, 16)] = z16
        return carry

    lax.fori_loop(0, GCH, _zrow, 0)
    for k in range(RPT // GCH):
        pltpu.sync_copy(rows0, shared_acc.at[pl.ds(sid * RPT + k * GCH, GCH)])
    pltpu.sync_copy(src_hbm.at[wid], src_v)
    pltpu.sync_copy(dst_hbm.at[wid], dst_v)
    plsc.subcore_barrier()

    def _sidx(idx_ref, j):
        return idx_ref.at[j // 2, pl.ds((j % 2) * GCH, GCH)]

    for b in range(NBUF - 1):
        pltpu.async_copy(hs_hbm.at[_sidx(src_v, b)], rows[b], semg[b])

    def _body(i, carry):
        for b in range(NBUF):
            j = NBUF * i + b
            o = (b + NBUF - 1) % NBUF

            @pl.when(j >= 1)
            def _wait_prev_scatter():
                pltpu.make_async_copy(rows[o], shared_acc.at[_sidx(dst_v, j)],
                                      sems[o]).wait()

            @pl.when(j + NBUF - 1 < NGC)
            def _issue_gather():
                pltpu.async_copy(hs_hbm.at[_sidx(src_v, j + NBUF - 1)],
                                 rows[o], semg[o])

            pltpu.make_async_copy(hs_hbm.at[_sidx(src_v, j)], rows[b],
                                  semg[b]).wait()
            pltpu.async_copy(rows[b], shared_acc.at[_sidx(dst_v, j)], sems[b],
                             add=True)
        return carry

    lax.fori_loop(0, NGC // NBUF, _body, 0)
    pltpu.make_async_copy(rows[(NGC - 1) % NBUF],
                          shared_acc.at[_sidx(dst_v, NGC - 1)],
                          sems[(NGC - 1) % NBUF]).wait()
    plsc.subcore_barrier()

    for k in range(RPT // CHUNK):
        r = sid * RPT + k * CHUNK
        pltpu.sync_copy(shared_acc.at[pl.ds(r, CHUNK)],
                        out_hbm.at[cid, pl.ds(r, CHUNK)])


_RB = 2560
_NBLK = NP // _RB


def _dis_block(deg_ref):
    return lax.rsqrt(deg_ref[0] + deg_ref[1] + 1.0)


def _tc1_body(x_ref, w_ref, deg_ref, hs_ref):
    dis = _dis_block(deg_ref)
    h = jnp.dot(x_ref[...], w_ref[...], preferred_element_type=jnp.float32)
    hs_ref[...] = h * jnp.broadcast_to(dis, (_RB, H))


_tc1 = pl.pallas_call(
    _tc1_body,
    grid=(_NBLK,),
    in_specs=[
        pl.BlockSpec((_RB, D), lambda i: (i, 0)),
        pl.BlockSpec((D, H), lambda i: (0, 0)),
        pl.BlockSpec((2, _RB, 1), lambda i: (0, i, 0)),
    ],
    out_specs=pl.BlockSpec((_RB, H), lambda i: (i, 0)),
    out_shape=jax.ShapeDtypeStruct((NP, H), jnp.float32),
)


def _tc_mid_body(agg_ref, hsp_ref, deg_ref, b_ref, w_ref, out_ref):
    disb = jnp.broadcast_to(_dis_block(deg_ref), (_RB, H))
    t = (agg_ref[0] + agg_ref[1] + hsp_ref[...]) * disb + b_ref[...]
    t = jnp.where(t >= 0, t, 0.01 * t)
    out_ref[...] = jnp.dot(t, w_ref[...], preferred_element_type=jnp.float32) * disb


_tc_mid = pl.pallas_call(
    _tc_mid_body,
    grid=(_NBLK,),
    in_specs=[
        pl.BlockSpec((2, _RB, H), lambda i: (0, i, 0)),
        pl.BlockSpec((_RB, H), lambda i: (i, 0)),
        pl.BlockSpec((2, _RB, 1), lambda i: (0, i, 0)),
        pl.BlockSpec((1, H), lambda i: (0, 0)),
        pl.BlockSpec((H, H), lambda i: (0, 0)),
    ],
    out_specs=pl.BlockSpec((_RB, H), lambda i: (i, 0)),
    out_shape=jax.ShapeDtypeStruct((NP, H), jnp.float32),
)


def _tc_final_body(agg_ref, hsp_ref, deg_ref, b_ref, batch_ref,
                   wfc_ref, bfc_ref, out_ref):
    dis = lax.rsqrt(deg_ref[0] + deg_ref[1] + 1.0)
    h3 = ((agg_ref[0] + agg_ref[1] + hsp_ref[...])
          * jnp.broadcast_to(dis, (NP, H)) + b_ref[...])
    bt = batch_ref[...]
    iot = lax.broadcasted_iota(jnp.int32, (B, NP), 0)
    onehot = jnp.where(bt == iot, 1.0, 0.0).astype(jnp.float32)
    sums = jnp.dot(onehot, h3, preferred_element_type=jnp.float32)
    cnt = jnp.sum(onehot, axis=1, keepdims=True)
    pooled = sums / jnp.maximum(cnt, 1.0)
    embed = jnp.dot(pooled, wfc_ref[...],
                    preferred_element_type=jnp.float32) + bfc_ref[...]
    m = jnp.max(embed, axis=1, keepdims=True)
    ex = jnp.exp(embed - m)
    lse = jnp.log(jnp.sum(ex, axis=1, keepdims=True)) + m
    out_ref[...] = embed - lse


_tc_final = pl.pallas_call(
    _tc_final_body,
    out_shape=jax.ShapeDtypeStruct((B, C), jnp.float32),
)


def kernel(x, edge_index, batch, W1, b1, W2, b2, W3, b3, Wfc, bfc):
    pad = jnp.arange(EP - E, dtype=jnp.int32)
    src_p = jnp.concatenate([edge_index[0].astype(jnp.int32), pad % N])
    dst_p = jnp.concatenate([edge_index[1].astype(jnp.int32),
                             pad % (NP - N) + N])
    src_p = src_p.reshape(NW, NCH, CHUNK)
    dst_p = dst_p.reshape(NW, NCH, CHUNK)
    x_p = jnp.pad(x, ((0, NP - N), (0, 0)))
    batch_p = jnp.pad(batch, (0, NP - N), constant_values=-1).reshape(1, NP)

    deg2 = _sc_degree(dst_p).reshape(2, NP, 1)
    hs1 = _tc1(x_p, W1, deg2)
    agg1 = _sc_aggregate(hs1, src_p, dst_p)
    hs2 = _tc_mid(agg1, hs1, deg2, b1.reshape(1, H), W2)
    agg2 = _sc_aggregate(hs2, src_p, dst_p)
    hs3 = _tc_mid(agg2, hs2, deg2, b2.reshape(1, H), W3)
    agg3 = _sc_aggregate(hs3, src_p, dst_p)
    logits = _tc_final(agg3, hs3, deg2, b3.reshape(1, H), batch_p,
                       Wfc, bfc.reshape(1, C))
    return logits
